# Initial kernel scaffold; baseline (speedup 1.0000x reference)
import jax
import jax.numpy as jnp
from jax.experimental import pallas as pl


def kernel(x, means, log_pi):
    raise NotImplementedError("write your pallas kernel here")


# speedup vs baseline: 2.0563x; 2.0563x over previous
import jax
import jax.numpy as jnp
import numpy as np
from jax.experimental import pallas as pl
from jax.experimental.pallas import tpu as pltpu

N, K, D = 16384, 1024, 256
BN = 256
BLOCKS_PER_CHUNK = 1024 // BN
_C0 = np.float32(0.5 * D) * np.float32(np.log(np.float32(2.0 * np.pi)))


def _estep_kernel(x_ref, means_ref, log_pi_ref, m2_ref,
                  l_ref, zsum_ref, s_ref, acc8_ref):
    i = pl.program_id(0)
    xb = x_ref[...]
    means = means_ref[...]
    x2 = jnp.sum(xb * xb, axis=1, keepdims=True)
    xm = jax.lax.dot_general(xb.astype(jnp.bfloat16), means.astype(jnp.bfloat16),
                             (((1,), (1,)), ((), ())),
                             preferred_element_type=jnp.float32)
    dist2 = x2 - 2.0 * xm + m2_ref[...]
    logp = log_pi_ref[...] - 0.5 * dist2 - _C0
    m = jnp.max(logp, axis=1, keepdims=True)
    e = jnp.exp(logp - m)
    s = jnp.sum(e, axis=1, keepdims=True)
    l_ref[...] = m + jnp.log(s)
    z = e / s

    @pl.when(i == 0)
    def _init():
        zsum_ref[...] = jnp.zeros_like(zsum_ref)
        s_ref[...] = jnp.zeros_like(s_ref)

    @pl.when(i % BLOCKS_PER_CHUNK == 0)
    def _chunk_init():
        acc8_ref[...] = jnp.zeros_like(acc8_ref)

    acc = acc8_ref[...]
    for t in range(BN // 8):
        acc = acc + jax.lax.slice(z, (8 * t, 0), (8 * t + 8, K))
    acc8_ref[...] = acc

    @pl.when(i % BLOCKS_PER_CHUNK == BLOCKS_PER_CHUNK - 1)
    def _chunk_fini():
        a8 = acc8_ref[...]
        a4 = jax.lax.slice(a8, (0, 0), (4, K)) + jax.lax.slice(a8, (4, 0), (8, K))
        a2 = jax.lax.slice(a4, (0, 0), (2, K)) + jax.lax.slice(a4, (2, 0), (4, K))
        a1 = jax.lax.slice(a2, (0, 0), (1, K)) + jax.lax.slice(a2, (1, 0), (2, K))
        zsum_ref[...] += a1

    s_ref[...] += jax.lax.dot_general(z.astype(jnp.bfloat16), xb.astype(jnp.bfloat16),
                                      (((0,), (0,)), ((), ())),
                                      preferred_element_type=jnp.float32)


def _sort_kernel(zsum_ref, means_ref, log_pi_ref, m2_ref, s_ref,
                 mask_ref, lp_perm_ref, m2_perm_ref, means_perm_ref,
                 new_means_ref, new_pi_ref):
    zs = zsum_ref[...]
    zi = zs.reshape(K, 1)
    ii = jax.lax.broadcasted_iota(jnp.int32, (K, K), 0)
    jj = jax.lax.broadcasted_iota(jnp.int32, (K, K), 1)
    cmp = (zi > zs) | ((zi == zs) & (ii > jj))
    rank = jnp.sum(cmp.astype(jnp.int32), axis=0)[None, :]
    perm = (ii == rank).astype(jnp.float32)

    zsum_perm = jnp.sum(perm * zs, axis=1)[None, :]
    mask = (zsum_perm > 2.0).astype(jnp.float32)
    mask_ref[...] = mask
    lp_perm_ref[...] = jnp.sum(perm * log_pi_ref[...], axis=1)[None, :]
    m2_perm_ref[...] = jnp.sum(perm * m2_ref[...], axis=1)[None, :]
    means_perm_ref[...] = jax.lax.dot_general(
        perm, means_ref[...], (((1,), (0,)), ((), ())),
        preferred_element_type=jnp.float32, precision=jax.lax.Precision.HIGHEST)
    s_perm = jax.lax.dot_general(
        perm, s_ref[...], (((1,), (0,)), ((), ())),
        preferred_element_type=jnp.float32, precision=jax.lax.Precision.HIGHEST)
    zsk = zsum_perm * mask
    new_pi_ref[...] = zsk / (jnp.sum(zsk) + 1e-8)
    new_means_ref[...] = (s_perm * mask.reshape(K, 1)) / (zsk.reshape(K, 1) + 1e-8)


def _zout_kernel(x_ref, means_perm_ref, lp_perm_ref, m2_perm_ref, mask_ref,
                 l_ref, z_ref):
    xb = x_ref[...]
    means = means_perm_ref[...]
    x2 = jnp.sum(xb * xb, axis=1, keepdims=True)
    xm = jax.lax.dot_general(xb.astype(jnp.bfloat16), means.astype(jnp.bfloat16),
                             (((1,), (1,)), ((), ())),
                             preferred_element_type=jnp.float32)
    logp = lp_perm_ref[...] - 0.5 * (x2 - 2.0 * xm + m2_perm_ref[...]) - _C0
    z_ref[...] = jnp.exp(logp - l_ref[...]) * mask_ref[...]


@jax.jit
def kernel(x, means, log_pi):
    log_pi2 = log_pi.reshape(1, K)
    m2 = jnp.sum(means * means, axis=1).reshape(1, K)
    nb = N // BN

    l2, zsum, s = pl.pallas_call(
        _estep_kernel,
        grid=(nb,),
        in_specs=[
            pl.BlockSpec((BN, D), lambda i: (i, 0)),
            pl.BlockSpec((K, D), lambda i: (0, 0)),
            pl.BlockSpec((1, K), lambda i: (0, 0)),
            pl.BlockSpec((1, K), lambda i: (0, 0)),
        ],
        out_specs=[
            pl.BlockSpec((BN, 1), lambda i: (i, 0)),
            pl.BlockSpec((1, K), lambda i: (0, 0)),
            pl.BlockSpec((K, D), lambda i: (0, 0)),
        ],
        out_shape=[
            jax.ShapeDtypeStruct((N, 1), jnp.float32),
            jax.ShapeDtypeStruct((1, K), jnp.float32),
            jax.ShapeDtypeStruct((K, D), jnp.float32),
        ],
        scratch_shapes=[pltpu.VMEM((8, K), jnp.float32)],
    )(x, means, log_pi2, m2)

    mask, lp_perm, m2_perm, means_perm, new_means, new_pi = pl.pallas_call(
        _sort_kernel,
        out_shape=[
            jax.ShapeDtypeStruct((1, K), jnp.float32),
            jax.ShapeDtypeStruct((1, K), jnp.float32),
            jax.ShapeDtypeStruct((1, K), jnp.float32),
            jax.ShapeDtypeStruct((K, D), jnp.float32),
            jax.ShapeDtypeStruct((K, D), jnp.float32),
            jax.ShapeDtypeStruct((1, K), jnp.float32),
        ],
    )(zsum, means, log_pi2, m2, s)

    z = pl.pallas_call(
        _zout_kernel,
        grid=(nb,),
        in_specs=[
            pl.BlockSpec((BN, D), lambda i: (i, 0)),
            pl.BlockSpec((K, D), lambda i: (0, 0)),
            pl.BlockSpec((1, K), lambda i: (0, 0)),
            pl.BlockSpec((1, K), lambda i: (0, 0)),
            pl.BlockSpec((1, K), lambda i: (0, 0)),
            pl.BlockSpec((BN, 1), lambda i: (i, 0)),
        ],
        out_specs=pl.BlockSpec((BN, K), lambda i: (i, 0)),
        out_shape=jax.ShapeDtypeStruct((N, K), jnp.float32),
    )(x, means_perm, lp_perm, m2_perm, mask, l2)

    return l2.reshape(N), z, new_means, new_pi.reshape(K)
